# baseline (speedup 1.0000x reference)
import functools

import jax
import jax.numpy as jnp
from jax import lax
from jax.experimental import pallas as pl
from jax.experimental.pallas import tpu as pltpu

POOL_LVLS = 3
POOL_DIVS = 4
NWIN = sum(POOL_DIVS ** l for l in range(POOL_LVLS))

T = 4096
C = 128
NB8 = T // 8
NB64 = T // 64

NEG_INF = float("-inf")


def _window_bounds(L, level, div_index):
    ndiv = POOL_DIVS ** level
    div_length = (L + (ndiv - 1)) // ndiv
    if ndiv <= 1:
        div_start = jnp.zeros((), jnp.int32)
    else:
        q = ndiv - 1
        r = (L - div_length) * div_index
        quotient = r // q
        rem = r - quotient * q
        div_start = quotient + (2 * rem > q).astype(jnp.int32)
    return div_start, div_length


def _spp_kernel(b0, len_ref, x_ref, out_ref, bm8_ref, bm64_ref):
    b = pl.program_id(0) + b0
    x = x_ref[0]

    bm8 = jnp.max(x.reshape(NB8, 8, C), axis=1)
    bm8_ref[...] = bm8
    bm64_ref[...] = jnp.max(bm8.reshape(NB64, 8, C), axis=1)

    L = len_ref[b]

    row8 = lax.broadcasted_iota(jnp.int32, (8, C), 0)
    j64 = lax.broadcasted_iota(jnp.int32, (NB64, C), 0)

    w = 0
    for level in range(POOL_LVLS):
        for div_index in range(POOL_DIVS ** level):
            s, dl = _window_bounds(L, level, div_index)
            e = s + dl

            kh = s // 8
            kt = (e - 1) // 8
            jh = kh // 8
            jt = kt // 8

            head = x_ref[0, pl.ds(kh * 8, 8), :]
            hpos = kh * 8 + row8
            head = jnp.where((hpos >= s) & (hpos < e), head, NEG_INF)
            tail = x_ref[0, pl.ds(kt * 8, 8), :]
            tpos = kt * 8 + row8
            tail = jnp.where((tpos >= s) & (tpos < e), tail, NEG_INF)
            acc = jnp.maximum(jnp.max(head, axis=0), jnp.max(tail, axis=0))

            mh = bm8_ref[pl.ds(jh * 8, 8), :]
            mk = jh * 8 + row8
            mh = jnp.where((mk > kh) & (mk < kt), mh, NEG_INF)
            mt = bm8_ref[pl.ds(jt * 8, 8), :]
            mk2 = jt * 8 + row8
            mt = jnp.where((mk2 > kh) & (mk2 < kt), mt, NEG_INF)
            acc = jnp.maximum(acc, jnp.max(mh, axis=0))
            acc = jnp.maximum(acc, jnp.max(mt, axis=0))

            inner = jnp.where((j64 > jh) & (j64 < jt), bm64_ref[...], NEG_INF)
            acc = jnp.maximum(acc, jnp.max(inner, axis=0))

            out_ref[0, w, :] = acc
            w += 1


def _tc_call(sequences, lengths, b0, nb):
    return pl.pallas_call(
        functools.partial(_spp_kernel, b0),
        grid=(nb,),
        in_specs=[
            pl.BlockSpec(memory_space=pltpu.SMEM),
            pl.BlockSpec((1, T, C), lambda b: (b + b0, 0, 0)),
        ],
        out_specs=pl.BlockSpec((1, NWIN, C), lambda b: (b, 0, 0)),
        out_shape=jax.ShapeDtypeStruct((nb, NWIN, C), jnp.float32),
        scratch_shapes=[
            pltpu.VMEM((NB8, C), jnp.float32),
            pltpu.VMEM((NB64, C), jnp.float32),
        ],
    )(lengths, sequences)


from jax.experimental.pallas import tpu_sc as plsc

NLANE = 16
NWORK = 32
CG = C // NLANE


NCHUNK = 4
CHUNK = T // NCHUNK


def _sc_body(nb, x_hbm, len_hbm, out_hbm, lens_v, lens_vm, xv, bm8v, bm64v,
             outv, sems):
    ncores = 2
    wid = lax.axis_index("s") * ncores + lax.axis_index("c")
    tasks_per_worker = (nb * CG) // NWORK
    pltpu.sync_copy(len_hbm, lens_vm.at[pl.ds(0, 16)])
    neg = jnp.full((NLANE,), NEG_INF, jnp.float32)

    def task_body(i, _):
        t = wid * tasks_per_worker + i
        b = t // CG
        cg16 = (t % CG) * NLANE
        copies = [
            pltpu.async_copy(
                x_hbm.at[b, pl.ds(c * CHUNK, CHUNK), pl.ds(cg16, NLANE)],
                xv.at[pl.ds(c * CHUNK, CHUNK)],
                sems.at[c],
            )
            for c in range(NCHUNK)
        ]
        L = lens_vm[pl.ds(b, NLANE)][0]

        def p1(k, _c):
            base = k * 8
            m = xv[base, :]
            for j in range(1, 8):
                m = jnp.maximum(m, xv[base + j, :])
            bm8v[k, :] = m
            return 0

        for c in range(NCHUNK):
            copies[c].wait()
            lax.fori_loop(c * (NB8 // NCHUNK), (c + 1) * (NB8 // NCHUNK),
                          p1, 0, unroll=8)

        def p2(k, _c):
            base = k * 8
            m = bm8v[base, :]
            for j in range(1, 8):
                m = jnp.maximum(m, bm8v[base + j, :])
            bm64v[k, :] = m
            return 0

        lax.fori_loop(0, NB64, p2, 0, unroll=8)

        def blk64(k, a):
            return jnp.maximum(a, bm64v[k, :])

        w = 0
        for level in range(POOL_LVLS):
            ndiv = POOL_DIVS ** level
            for div_index in range(ndiv):
                if ndiv == 1:
                    s = jnp.zeros((), jnp.int32)
                    dl = L
                else:
                    dl = (L + (ndiv - 1)) // ndiv
                    q = ndiv - 1
                    r = (L - dl) * div_index
                    quo = r // q
                    rem = r - quo * q
                    s = quo + (2 * rem > q).astype(jnp.int32)
                e = s + dl
                kh = s // 8
                kt = (e - 1) // 8
                jh = kh // 8
                jt = kt // 8

                acc = neg
                for base_k in (kh, kt):
                    base = base_k * 8
                    for j in range(8):
                        tt = base + j
                        pred = (tt >= s) & (tt < e)
                        acc = jnp.where(pred, jnp.maximum(acc, xv[tt, :]), acc)
                for base_j in (jh, jt):
                    base = base_j * 8
                    for j in range(8):
                        kk = base + j
                        pred = (kk > kh) & (kk < kt)
                        acc = jnp.where(pred, jnp.maximum(acc, bm8v[kk, :]), acc)
                acc = lax.fori_loop(jh + 1, jt, blk64, acc)
                outv[w, :] = acc
                w += 1
        pltpu.sync_copy(outv, out_hbm.at[b, pl.ds(0, NWIN), pl.ds(cg16, NLANE)])
        return 0

    lax.fori_loop(0, tasks_per_worker, task_body, 0)


NWIN_PAD = 24


def _sc_call(sequences, lengths, nb):
    mesh = plsc.VectorSubcoreMesh(core_axis_name="c", subcore_axis_name="s")
    kfn = functools.partial(
        pl.kernel,
        mesh=mesh,
        out_type=jax.ShapeDtypeStruct((nb, NWIN_PAD, C), jnp.float32),
        scratch_types=[
            pltpu.SMEM((16,), jnp.int32),
            pltpu.VMEM((32,), jnp.int32),
            pltpu.VMEM((T, NLANE), jnp.float32),
            pltpu.VMEM((NB8, NLANE), jnp.float32),
            pltpu.VMEM((NB64, NLANE), jnp.float32),
            pltpu.VMEM((NWIN, NLANE), jnp.float32),
            pltpu.SemaphoreType.DMA((NCHUNK,)),
        ],
        compiler_params=pltpu.CompilerParams(use_tc_tiling_on_sc=False),
    )(functools.partial(_sc_body, nb))
    return kfn(sequences, lengths)[:, :NWIN, :]


B_SC = 4


@jax.jit
def kernel(sequences, lengths):
    B = sequences.shape[0]
    if B_SC == 0:
        return _tc_call(sequences, lengths, 0, B)
    out_sc = _sc_call(sequences, lengths, B_SC)
    if B_SC == B:
        return out_sc
    out_tc = _tc_call(sequences, lengths, B_SC, B - B_SC)
    return jnp.concatenate([out_sc, out_tc], axis=0)


# speedup vs baseline: 1.2017x; 1.2017x over previous
import functools

import jax
import jax.numpy as jnp
from jax import lax
from jax.experimental import pallas as pl
from jax.experimental.pallas import tpu as pltpu

POOL_LVLS = 3
POOL_DIVS = 4
NWIN = sum(POOL_DIVS ** l for l in range(POOL_LVLS))

T = 4096
C = 128
NB8 = T // 8
NB64 = T // 64

NEG_INF = float("-inf")


def _window_bounds(L, level, div_index):
    ndiv = POOL_DIVS ** level
    div_length = (L + (ndiv - 1)) // ndiv
    if ndiv <= 1:
        div_start = jnp.zeros((), jnp.int32)
    else:
        q = ndiv - 1
        r = (L - div_length) * div_index
        quotient = r // q
        rem = r - quotient * q
        div_start = quotient + (2 * rem > q).astype(jnp.int32)
    return div_start, div_length


def _spp_kernel(b0, len_ref, x_ref, out_ref, bm8_ref, bm64_ref):
    b = pl.program_id(0) + b0
    x = x_ref[0]

    bm8 = jnp.max(x.reshape(NB8, 8, C), axis=1)
    bm8_ref[...] = bm8
    bm64_ref[...] = jnp.max(bm8.reshape(NB64, 8, C), axis=1)

    L = len_ref[b]

    row8 = lax.broadcasted_iota(jnp.int32, (8, C), 0)
    j64 = lax.broadcasted_iota(jnp.int32, (NB64, C), 0)

    w = 0
    for level in range(POOL_LVLS):
        for div_index in range(POOL_DIVS ** level):
            s, dl = _window_bounds(L, level, div_index)
            e = s + dl

            kh = s // 8
            kt = (e - 1) // 8
            jh = kh // 8
            jt = kt // 8

            head = x_ref[0, pl.ds(kh * 8, 8), :]
            hpos = kh * 8 + row8
            head = jnp.where((hpos >= s) & (hpos < e), head, NEG_INF)
            tail = x_ref[0, pl.ds(kt * 8, 8), :]
            tpos = kt * 8 + row8
            tail = jnp.where((tpos >= s) & (tpos < e), tail, NEG_INF)
            acc = jnp.maximum(jnp.max(head, axis=0), jnp.max(tail, axis=0))

            mh = bm8_ref[pl.ds(jh * 8, 8), :]
            mk = jh * 8 + row8
            mh = jnp.where((mk > kh) & (mk < kt), mh, NEG_INF)
            mt = bm8_ref[pl.ds(jt * 8, 8), :]
            mk2 = jt * 8 + row8
            mt = jnp.where((mk2 > kh) & (mk2 < kt), mt, NEG_INF)
            acc = jnp.maximum(acc, jnp.max(mh, axis=0))
            acc = jnp.maximum(acc, jnp.max(mt, axis=0))

            inner = jnp.where((j64 > jh) & (j64 < jt), bm64_ref[...], NEG_INF)
            acc = jnp.maximum(acc, jnp.max(inner, axis=0))

            out_ref[0, w, :] = acc
            w += 1


def _tc_call(sequences, lengths, b0, nb):
    return pl.pallas_call(
        functools.partial(_spp_kernel, b0),
        grid=(nb,),
        in_specs=[
            pl.BlockSpec(memory_space=pltpu.SMEM),
            pl.BlockSpec((1, T, C), lambda b: (b + b0, 0, 0)),
        ],
        out_specs=pl.BlockSpec((1, NWIN, C), lambda b: (b, 0, 0)),
        out_shape=jax.ShapeDtypeStruct((nb, NWIN, C), jnp.float32),
        scratch_shapes=[
            pltpu.VMEM((NB8, C), jnp.float32),
            pltpu.VMEM((NB64, C), jnp.float32),
        ],
    )(lengths, sequences)


from jax.experimental.pallas import tpu_sc as plsc

NLANE = 16
NWORK = 32
CG = C // NLANE


NCHUNK = 4
CHUNK = T // NCHUNK


def _sc_body(nb, x_hbm, len_hbm, out_hbm, lens_v, lens_vm, xv, bm8v, bm64v,
             outv, sems):
    ncores = 2
    wid = lax.axis_index("s") * ncores + lax.axis_index("c")
    tasks_per_worker = (nb * CG) // NWORK
    pltpu.sync_copy(len_hbm, lens_vm.at[pl.ds(0, 16)])
    neg = jnp.full((NLANE,), NEG_INF, jnp.float32)

    def task_body(i, _):
        t = wid * tasks_per_worker + i
        b = t // CG
        cg16 = (t % CG) * NLANE
        copies = [
            pltpu.async_copy(
                x_hbm.at[b, pl.ds(c * CHUNK, CHUNK), pl.ds(cg16, NLANE)],
                xv.at[pl.ds(c * CHUNK, CHUNK)],
                sems.at[c],
            )
            for c in range(NCHUNK)
        ]
        L = lens_vm[pl.ds(b, NLANE)][0]

        def p1(k, _c):
            base = k * 8
            m = xv[base, :]
            for j in range(1, 8):
                m = jnp.maximum(m, xv[base + j, :])
            bm8v[k, :] = m
            return 0

        for c in range(NCHUNK):
            copies[c].wait()
            lax.fori_loop(c * (NB8 // NCHUNK), (c + 1) * (NB8 // NCHUNK),
                          p1, 0, unroll=8)

        def p2(k, _c):
            base = k * 8
            m = bm8v[base, :]
            for j in range(1, 8):
                m = jnp.maximum(m, bm8v[base + j, :])
            bm64v[k, :] = m
            return 0

        lax.fori_loop(0, NB64, p2, 0, unroll=8)

        def blk64(k, a):
            return jnp.maximum(a, bm64v[k, :])

        def wbody(w, _c):
            lvl1 = w < 5
            lvl0 = w == 0
            ndiv_m1 = jnp.where(lvl0, 0, jnp.where(lvl1, 3, 15))
            inv_ndiv = jnp.where(lvl0, 1.0, jnp.where(lvl1, 0.25, 0.0625))
            d = w - jnp.where(lvl0, 0, jnp.where(lvl1, 1, 5))
            dl = ((L + ndiv_m1).astype(jnp.float32) * inv_ndiv)
            dl = dl.astype(jnp.int32)
            q = ndiv_m1
            inv_q = jnp.where(lvl1, 1.0 / 3.0, 1.0 / 15.0)
            r = (L - dl) * d
            quo = (r.astype(jnp.float32) * inv_q).astype(jnp.int32)
            quo = jnp.where(lvl0, 0, quo)
            rem = r - quo * q
            s = quo + (2 * rem > q).astype(jnp.int32)
            e = s + dl

            kh = s >> 3
            kt = (e - 1) >> 3
            jh = kh >> 3
            jt = kt >> 3

            acc = neg
            for base_k in (kh, kt):
                base = base_k * 8
                for j in range(8):
                    tt = base + j
                    pred = (tt >= s) & (tt < e)
                    acc = jnp.where(pred, jnp.maximum(acc, xv[tt, :]), acc)
            for base_j in (jh, jt):
                base = base_j * 8
                for j in range(8):
                    kk = base + j
                    pred = (kk > kh) & (kk < kt)
                    acc = jnp.where(pred, jnp.maximum(acc, bm8v[kk, :]), acc)
            acc = lax.fori_loop(jh + 1, jt, blk64, acc)
            outv[w, :] = acc
            return 0

        lax.fori_loop(0, NWIN, wbody, 0)
        pltpu.sync_copy(outv, out_hbm.at[b, pl.ds(0, NWIN), pl.ds(cg16, NLANE)])
        return 0

    lax.fori_loop(0, tasks_per_worker, task_body, 0)


NWIN_PAD = 24


def _sc_call(sequences, lengths, nb):
    mesh = plsc.VectorSubcoreMesh(core_axis_name="c", subcore_axis_name="s")
    kfn = functools.partial(
        pl.kernel,
        mesh=mesh,
        out_type=jax.ShapeDtypeStruct((nb, NWIN_PAD, C), jnp.float32),
        scratch_types=[
            pltpu.SMEM((16,), jnp.int32),
            pltpu.VMEM((32,), jnp.int32),
            pltpu.VMEM((T, NLANE), jnp.float32),
            pltpu.VMEM((NB8, NLANE), jnp.float32),
            pltpu.VMEM((NB64, NLANE), jnp.float32),
            pltpu.VMEM((NWIN, NLANE), jnp.float32),
            pltpu.SemaphoreType.DMA((NCHUNK,)),
        ],
        compiler_params=pltpu.CompilerParams(use_tc_tiling_on_sc=False),
    )(functools.partial(_sc_body, nb))
    return kfn(sequences, lengths)[:, :NWIN, :]


B_SC = 4


@jax.jit
def kernel(sequences, lengths):
    B = sequences.shape[0]
    if B_SC == 0:
        return _tc_call(sequences, lengths, 0, B)
    out_sc = _sc_call(sequences, lengths, B_SC)
    if B_SC == B:
        return out_sc
    out_tc = _tc_call(sequences, lengths, B_SC, B - B_SC)
    return jnp.concatenate([out_sc, out_tc], axis=0)
